# baseline (speedup 1.0000x reference)
import functools

import jax
import jax.numpy as jnp
from jax import lax
from jax.experimental import pallas as pl
from jax.experimental.pallas import tpu as pltpu
from jax.experimental.pallas import tpu_sc as plsc

EMBED = 64
T = 2
NC, NS = 2, 16
NW = NC * NS
NREG = EMBED // 16
STREAM = 40

_SC_PARAMS = pltpu.CompilerParams(
    use_tc_tiling_on_sc=True, needs_layout_passes=False)


def _mesh():
    return plsc.VectorSubcoreMesh(
        core_axis_name="c", subcore_axis_name="s",
        num_cores=NC, num_subcores=NS)


def _make_transpose(V):
    VP = V // 2
    NB = V // 128
    TAIL = V - NB * 128

    @functools.partial(
        pl.kernel,
        out_type=jax.ShapeDtypeStruct((VP, 128), jnp.float32),
        mesh=_mesh(),
        compiler_params=_SC_PARAMS,
        scratch_types=[
            pltpu.VMEM((EMBED, 128), jnp.float32),
            pltpu.VMEM((EMBED, 128), jnp.float32),
            pltpu.VMEM((EMBED, 128), jnp.float32),
            pltpu.VMEM((EMBED, 128), jnp.float32),
            pltpu.SemaphoreType.DMA,
            pltpu.SemaphoreType.DMA,
        ],
    )
    def tr_kernel(tab_hbm, tail_hbm, scr_hbm, in0, in1, out0, out1,
                  sem_i, sem_o):
        wid = lax.axis_index("s") * NC + lax.axis_index("c")
        nblk = 244 + jnp.where(wid < NB % NW, 1, 0)
        ins = (in0, in1)
        outs = (out0, out1)
        lanes = lax.iota(jnp.int32, 16)

        def fire_in(bi, buf):
            v0 = (wid + NW * bi) * 128
            pltpu.make_async_copy(
                tab_hbm.at[:, pl.ds(v0, 128)], buf, sem_i).start()

        def wait_in(buf):
            pltpu.make_async_copy(
                tab_hbm.at[:, pl.ds(0, 128)], buf, sem_i).wait()

        def fire_out(bi, buf):
            b = wid + NW * bi
            pltpu.make_async_copy(
                buf, scr_hbm.at[pl.ds(64 * b, 64), :], sem_o).start()

        def wait_out(buf):
            pltpu.make_async_copy(
                buf, scr_hbm.at[pl.ds(0, 64), :], sem_o).wait()

        def transpose_block(src, dst):
            def jbody(j, carry):
                for h in range(2):
                    col = jnp.full((16,), 2 * j + h, jnp.int32)
                    for k in range(NREG):
                        vec = plsc.load_gather(src, [lanes + 16 * k, col])
                        dst[j, pl.ds(64 * h + 16 * k, 16)] = vec
                return carry
            lax.fori_loop(0, 64, jbody, 0, unroll=8)

        fire_in(0, in0)
        fire_in(1, in1)

        def outer(ii, carry):
            for p in range(2):
                bi = 2 * ii + p
                wait_in(ins[p])

                @pl.when(ii >= 1)
                def _():
                    wait_out(outs[p])

                transpose_block(ins[p], outs[p])
                fire_out(bi, outs[p])

                @pl.when(bi + 2 < nblk)
                def _():
                    fire_in(bi + 2, ins[p])

            return carry

        lax.fori_loop(0, 122, outer, 0)

        @pl.when(nblk == 245)
        def _():
            wait_in(ins[0])
            wait_out(outs[0])
            transpose_block(ins[0], outs[0])
            fire_out(244, outs[0])

        wait_out(outs[1])
        wait_out(outs[0])

        @pl.when(wid == 5)
        def _():
            pltpu.sync_copy(tail_hbm, out0.at[pl.ds(0, TAIL // 2), :])
            pltpu.sync_copy(out0.at[pl.ds(0, TAIL // 2), :],
                            scr_hbm.at[pl.ds(64 * NB, TAIL // 2), :])

    return tr_kernel


def _make_gather_ln(B, S, VP):
    ROWS = B * S
    RPW = ROWS // NW
    NSEQ = RPW // S
    NSTR = S // STREAM
    SP = S // 2

    @functools.partial(
        pl.kernel,
        out_type=jax.ShapeDtypeStruct((SP, B, 128), jnp.float32),
        mesh=_mesh(),
        compiler_params=_SC_PARAMS,
        scratch_types=[
            pltpu.VMEM((RPW,), jnp.int32),
            pltpu.VMEM((RPW,), jnp.int32),
            pltpu.VMEM((RPW + 16,), jnp.int32),
            pltpu.VMEM((S, EMBED), jnp.float32),
            pltpu.VMEM((EMBED,), jnp.float32),
            pltpu.VMEM((EMBED,), jnp.float32),
            pltpu.VMEM((S, 128), jnp.float32),
            pltpu.VMEM((S, 128), jnp.float32),
            pltpu.VMEM((SP, 128), jnp.float32),
            pltpu.VMEM((SP, 128), jnp.float32),
            pltpu.SemaphoreType.DMA,
            pltpu.SemaphoreType.DMA,
        ],
    )
    def gl_kernel(x_hbm, scr_hbm, gam_hbm, bet_hbm, poe_hbm, y_hbm,
                  raw_v, idx_v, off_v, poe_v, gam_v, bet_v,
                  g0, g1, o0, o1, sem_g, sem_o):
        wid = lax.axis_index("s") * NC + lax.axis_index("c")
        base_tok = wid * RPW
        base_seq = wid * NSEQ
        pltpu.sync_copy(x_hbm.at[pl.ds(base_tok, RPW)], raw_v)
        pltpu.sync_copy(poe_hbm, poe_v)
        pltpu.sync_copy(gam_hbm, gam_v)
        pltpu.sync_copy(bet_hbm, bet_v)

        def prep(i, carry):
            v = raw_v[pl.ds(16 * i, 16)]
            idx_v[pl.ds(16 * i, 16)] = v >> 1
            off_v[pl.ds(16 * i, 16)] = (v & 1) << 6
            return carry
        lax.fori_loop(0, RPW // 16, prep, 0)

        gam = [gam_v[pl.ds(16 * j, 16)] for j in range(NREG)]
        bet = [bet_v[pl.ds(16 * j, 16)] for j in range(NREG)]
        lanes = lax.iota(jnp.int32, 16)
        perms = [lanes ^ k for k in (1, 2, 4, 8)]
        gbuf = (g0, g1)
        obuf = (o0, o1)

        def allsum(v):
            for p in perms:
                v = v + v.at[p].get(mode="promise_in_bounds")
            return v

        def fire_gather(c, g):
            for k in range(NSTR):
                pltpu.make_async_copy(
                    scr_hbm.at[idx_v.at[pl.ds(c * S + k * STREAM, STREAM)]],
                    g.at[pl.ds(k * STREAM, STREAM), :], sem_g).start()

        def wait_gather(g):
            for k in range(NSTR):
                pltpu.make_async_copy(
                    scr_hbm.at[idx_v.at[pl.ds(0, STREAM)]],
                    g.at[pl.ds(k * STREAM, STREAM), :], sem_g).wait()

        def fire_out(c, o):
            pltpu.make_async_copy(o, y_hbm.at[:, base_seq + c, :], sem_o).start()

        def wait_out(o):
            pltpu.make_async_copy(o, y_hbm.at[:, 0, :], sem_o).wait()

        def compute(c, g, o):
            def group(gi, carry):
                off16 = off_v[pl.ds(c * S + 8 * gi, 16)]
                for u in range(8):
                    r = 8 * gi + u
                    off = off16[u]
                    x = [g[r, pl.ds(off + 16 * j, 16)]
                         + poe_v[r, pl.ds(16 * j, 16)] for j in range(NREG)]
                    tot = allsum((x[0] + x[1]) + (x[2] + x[3]))
                    tot2 = allsum((x[0] * x[0] + x[1] * x[1])
                                  + (x[2] * x[2] + x[3] * x[3]))
                    mean = tot * (1.0 / EMBED)
                    v = tot2 * (1.0 / EMBED) - mean * mean + 1e-5
                    ib = lax.bitcast_convert_type(v[0], jnp.int32)
                    ib = jnp.int32(0x5F3759DF) - (ib >> 1)
                    y = jnp.full(
                        (16,), lax.bitcast_convert_type(ib, jnp.float32),
                        jnp.float32)
                    for _ in range(3):
                        y = y * (1.5 - 0.5 * v * y * y)
                    orow = 4 * gi + u // 2
                    ocol = (u % 2) * 64
                    for j in range(NREG):
                        a = gam[j] * y
                        b = bet[j] - mean * a
                        o[orow, pl.ds(ocol + 16 * j, 16)] = x[j] * a + b
                return carry
            lax.fori_loop(0, S // 8, group, 0, unroll=2)

        fire_gather(0, g0)
        fire_gather(1, g1)

        def outer(cc, carry):
            for p in range(2):
                c = 2 * cc + p
                wait_gather(gbuf[p])

                @pl.when(cc >= 1)
                def _():
                    wait_out(obuf[p])

                compute(c, gbuf[p], obuf[p])
                fire_out(c, obuf[p])

                @pl.when(cc < NSEQ // 2 - 1)
                def _():
                    fire_gather(c + 2, gbuf[p])

            return carry

        lax.fori_loop(0, NSEQ // 2, outer, 0)
        wait_out(o0)
        wait_out(o1)

    return gl_kernel


def _make_finish(B, S):
    SP = S // 2

    def fin_kernel(y_ref, o_ref):
        for q in range(4):
            y = y_ref[q]
            a = y[:, 0:EMBED].T
            b = y[:, EMBED:128].T
            for t in range(T):
                o_ref[t, 2 * q] = a
                o_ref[t, 2 * q + 1] = b

    return pl.pallas_call(
        fin_kernel,
        grid=(SP // 4,),
        in_specs=[pl.BlockSpec((4, B, 128), lambda i: (i, 0, 0))],
        out_specs=pl.BlockSpec((T, 8, EMBED, B), lambda i: (0, i, 0, 0)),
        out_shape=jax.ShapeDtypeStruct((T, S, EMBED, B), jnp.float32),
    )


def kernel(x, emb_table, ln_gamma, ln_beta, poe):
    B, S = x.shape
    V = emb_table.shape[0]
    tab_t = emb_table.T
    nb = V // 128
    tail = emb_table[nb * 128:].reshape(-1, 128)
    xf = x.astype(jnp.int32).reshape(-1)
    scratch = _make_transpose(V)(tab_t, tail)
    y = _make_gather_ln(B, S, V // 2)(
        xf, scratch, ln_gamma, ln_beta, poe[:S])
    p = _make_finish(B, S)(y)
    return jnp.transpose(p, (0, 3, 1, 2))


# speedup vs baseline: 1.5727x; 1.5727x over previous
import functools

import jax
import jax.numpy as jnp
from jax import lax
from jax.experimental import pallas as pl
from jax.experimental.pallas import tpu as pltpu
from jax.experimental.pallas import tpu_sc as plsc

EMBED = 64
T = 2
NC, NS = 2, 16
NW = NC * NS
NREG = EMBED // 16
STREAM = 40

_SC_PARAMS = pltpu.CompilerParams(
    use_tc_tiling_on_sc=True, needs_layout_passes=False)


def _mesh():
    return plsc.VectorSubcoreMesh(
        core_axis_name="c", subcore_axis_name="s",
        num_cores=NC, num_subcores=NS)


def _make_transpose(V):
    VP = V // 2
    NB = V // 128
    TAIL = V - NB * 128

    @functools.partial(
        pl.kernel,
        out_type=jax.ShapeDtypeStruct((VP, 128), jnp.float32),
        mesh=_mesh(),
        compiler_params=_SC_PARAMS,
        scratch_types=[
            pltpu.VMEM((EMBED, 128), jnp.float32),
            pltpu.VMEM((EMBED, 128), jnp.float32),
            pltpu.VMEM((EMBED, 128), jnp.float32),
            pltpu.VMEM((EMBED, 128), jnp.float32),
            pltpu.SemaphoreType.DMA,
            pltpu.SemaphoreType.DMA,
        ],
    )
    def tr_kernel(tab_hbm, tail_hbm, scr_hbm, in0, in1, out0, out1,
                  sem_i, sem_o):
        wid = lax.axis_index("s") * NC + lax.axis_index("c")
        nblk = 244 + jnp.where(wid < NB % NW, 1, 0)
        ins = (in0, in1)
        outs = (out0, out1)
        lanes = lax.iota(jnp.int32, 16)

        def fire_in(bi, buf):
            v0 = (wid + NW * bi) * 128
            pltpu.make_async_copy(
                tab_hbm.at[:, pl.ds(v0, 128)], buf, sem_i).start()

        def wait_in(buf):
            pltpu.make_async_copy(
                tab_hbm.at[:, pl.ds(0, 128)], buf, sem_i).wait()

        def fire_out(bi, buf):
            b = wid + NW * bi
            pltpu.make_async_copy(
                buf, scr_hbm.at[pl.ds(64 * b, 64), :], sem_o).start()

        def wait_out(buf):
            pltpu.make_async_copy(
                buf, scr_hbm.at[pl.ds(0, 64), :], sem_o).wait()

        def transpose_block(src, dst):
            @plsc.parallel_loop(0, 64, unroll=8)
            def _(j):
                for h in range(2):
                    col = jnp.full((16,), 2 * j + h, jnp.int32)
                    for k in range(NREG):
                        vec = plsc.load_gather(src, [lanes + 16 * k, col])
                        dst[j, pl.ds(64 * h + 16 * k, 16)] = vec

        fire_in(0, in0)
        fire_in(1, in1)

        def outer(ii, carry):
            for p in range(2):
                bi = 2 * ii + p
                wait_in(ins[p])

                @pl.when(ii >= 1)
                def _():
                    wait_out(outs[p])

                transpose_block(ins[p], outs[p])
                fire_out(bi, outs[p])

                @pl.when(bi + 2 < nblk)
                def _():
                    fire_in(bi + 2, ins[p])

            return carry

        lax.fori_loop(0, 122, outer, 0)

        @pl.when(nblk == 245)
        def _():
            wait_in(ins[0])
            wait_out(outs[0])
            transpose_block(ins[0], outs[0])
            fire_out(244, outs[0])

        wait_out(outs[1])
        wait_out(outs[0])

        @pl.when(wid == 5)
        def _():
            pltpu.sync_copy(tail_hbm, out0.at[pl.ds(0, TAIL // 2), :])
            pltpu.sync_copy(out0.at[pl.ds(0, TAIL // 2), :],
                            scr_hbm.at[pl.ds(64 * NB, TAIL // 2), :])

    return tr_kernel


def _make_gather_ln(B, S, VP):
    ROWS = B * S
    RPW = ROWS // NW
    NSEQ = RPW // S
    NSTR = S // STREAM
    SP = S // 2

    @functools.partial(
        pl.kernel,
        out_type=jax.ShapeDtypeStruct((SP, B, 128), jnp.float32),
        mesh=_mesh(),
        compiler_params=_SC_PARAMS,
        scratch_types=[
            pltpu.VMEM((RPW,), jnp.int32),
            pltpu.VMEM((RPW,), jnp.int32),
            pltpu.VMEM((RPW + 16,), jnp.int32),
            pltpu.VMEM((S, EMBED), jnp.float32),
            pltpu.VMEM((EMBED,), jnp.float32),
            pltpu.VMEM((EMBED,), jnp.float32),
            pltpu.VMEM((S, 128), jnp.float32),
            pltpu.VMEM((S, 128), jnp.float32),
            pltpu.VMEM((SP, 128), jnp.float32),
            pltpu.VMEM((SP, 128), jnp.float32),
            pltpu.SemaphoreType.DMA,
            pltpu.SemaphoreType.DMA,
        ],
    )
    def gl_kernel(x_hbm, scr_hbm, gam_hbm, bet_hbm, poe_hbm, y_hbm,
                  raw_v, idx_v, off_v, poe_v, gam_v, bet_v,
                  g0, g1, o0, o1, sem_g, sem_o):
        wid = lax.axis_index("s") * NC + lax.axis_index("c")
        base_tok = wid * RPW
        base_seq = wid * NSEQ
        pltpu.sync_copy(x_hbm.at[pl.ds(base_tok, RPW)], raw_v)
        pltpu.sync_copy(poe_hbm, poe_v)
        pltpu.sync_copy(gam_hbm, gam_v)
        pltpu.sync_copy(bet_hbm, bet_v)

        def prep(i, carry):
            v = raw_v[pl.ds(16 * i, 16)]
            idx_v[pl.ds(16 * i, 16)] = v >> 1
            off_v[pl.ds(16 * i, 16)] = (v & 1) << 6
            return carry
        lax.fori_loop(0, RPW // 16, prep, 0)

        gam = [gam_v[pl.ds(16 * j, 16)] for j in range(NREG)]
        bet = [bet_v[pl.ds(16 * j, 16)] for j in range(NREG)]
        lanes = lax.iota(jnp.int32, 16)
        perms = [lanes ^ k for k in (1, 2, 4, 8)]
        gbuf = (g0, g1)
        obuf = (o0, o1)

        def allsum(v):
            for p in perms:
                v = v + v.at[p].get(mode="promise_in_bounds")
            return v

        def fire_gather(c, g):
            for k in range(NSTR):
                pltpu.make_async_copy(
                    scr_hbm.at[idx_v.at[pl.ds(c * S + k * STREAM, STREAM)]],
                    g.at[pl.ds(k * STREAM, STREAM), :], sem_g).start()

        def wait_gather(g):
            for k in range(NSTR):
                pltpu.make_async_copy(
                    scr_hbm.at[idx_v.at[pl.ds(0, STREAM)]],
                    g.at[pl.ds(k * STREAM, STREAM), :], sem_g).wait()

        def fire_out(c, o):
            pltpu.make_async_copy(o, y_hbm.at[:, base_seq + c, :], sem_o).start()

        def wait_out(o):
            pltpu.make_async_copy(o, y_hbm.at[:, 0, :], sem_o).wait()

        def compute(c, g, o):
            @plsc.parallel_loop(0, S // 8, unroll=2)
            def group(gi):
                off16 = off_v[pl.ds(c * S + 8 * gi, 16)]
                for u in range(8):
                    r = 8 * gi + u
                    off = off16[u]
                    x = [g[r, pl.ds(off + 16 * j, 16)]
                         + poe_v[r, pl.ds(16 * j, 16)] for j in range(NREG)]
                    tot = allsum((x[0] + x[1]) + (x[2] + x[3]))
                    tot2 = allsum((x[0] * x[0] + x[1] * x[1])
                                  + (x[2] * x[2] + x[3] * x[3]))
                    mean = tot * (1.0 / EMBED)
                    v = tot2 * (1.0 / EMBED) - mean * mean + 1e-5
                    ib = lax.bitcast_convert_type(v[0], jnp.int32)
                    ib = jnp.int32(0x5F3759DF) - (ib >> 1)
                    y = jnp.full(
                        (16,), lax.bitcast_convert_type(ib, jnp.float32),
                        jnp.float32)
                    for _ in range(3):
                        y = y * (1.5 - 0.5 * v * y * y)
                    orow = 4 * gi + u // 2
                    ocol = (u % 2) * 64
                    for j in range(NREG):
                        a = gam[j] * y
                        b = bet[j] - mean * a
                        o[orow, pl.ds(ocol + 16 * j, 16)] = x[j] * a + b

        fire_gather(0, g0)
        fire_gather(1, g1)

        def outer(cc, carry):
            for p in range(2):
                c = 2 * cc + p
                wait_gather(gbuf[p])

                @pl.when(cc >= 1)
                def _():
                    wait_out(obuf[p])

                compute(c, gbuf[p], obuf[p])
                fire_out(c, obuf[p])

                @pl.when(cc < NSEQ // 2 - 1)
                def _():
                    fire_gather(c + 2, gbuf[p])

            return carry

        lax.fori_loop(0, NSEQ // 2, outer, 0)
        wait_out(o0)
        wait_out(o1)

    return gl_kernel


def _make_finish(B, S):
    SP = S // 2

    def fin_kernel(y_ref, o_ref):
        for q in range(4):
            y = y_ref[q]
            a = y[:, 0:EMBED].T
            b = y[:, EMBED:128].T
            for t in range(T):
                o_ref[t, 2 * q] = a
                o_ref[t, 2 * q + 1] = b

    return pl.pallas_call(
        fin_kernel,
        grid=(SP // 4,),
        in_specs=[pl.BlockSpec((4, B, 128), lambda i: (i, 0, 0))],
        out_specs=pl.BlockSpec((T, 8, EMBED, B), lambda i: (0, i, 0, 0)),
        out_shape=jax.ShapeDtypeStruct((T, S, EMBED, B), jnp.float32),
    )


def kernel(x, emb_table, ln_gamma, ln_beta, poe):
    B, S = x.shape
    V = emb_table.shape[0]
    tab_t = emb_table.T
    nb = V // 128
    tail = emb_table[nb * 128:].reshape(-1, 128)
    xf = x.astype(jnp.int32).reshape(-1)
    scratch = _make_transpose(V)(tab_t, tail)
    y = _make_gather_ln(B, S, V // 2)(
        xf, scratch, ln_gamma, ln_beta, poe[:S])
    p = _make_finish(B, S)(y)
    return jnp.transpose(p, (0, 3, 1, 2))


# speedup vs baseline: 1.7509x; 1.1133x over previous
import functools

import jax
import jax.numpy as jnp
from jax import lax
from jax.experimental import pallas as pl
from jax.experimental.pallas import tpu as pltpu
from jax.experimental.pallas import tpu_sc as plsc

EMBED = 64
T = 2
NC, NS = 2, 16
NW = NC * NS
NREG = EMBED // 16
STREAM = 40

_SC_PARAMS = pltpu.CompilerParams(
    use_tc_tiling_on_sc=True, needs_layout_passes=False)


def _mesh():
    return plsc.VectorSubcoreMesh(
        core_axis_name="c", subcore_axis_name="s",
        num_cores=NC, num_subcores=NS)


def _make_transpose(V):
    VP = V // 2
    NB = V // 128
    TAIL = V - NB * 128

    @functools.partial(
        pl.kernel,
        out_type=jax.ShapeDtypeStruct((VP, 128), jnp.float32),
        mesh=_mesh(),
        compiler_params=_SC_PARAMS,
        scratch_types=[
            pltpu.VMEM((EMBED, 128), jnp.float32),
            pltpu.VMEM((EMBED, 128), jnp.float32),
            pltpu.VMEM((EMBED, 128), jnp.float32),
            pltpu.VMEM((EMBED, 128), jnp.float32),
            pltpu.SemaphoreType.DMA,
            pltpu.SemaphoreType.DMA,
        ],
    )
    def tr_kernel(tab_hbm, tail_hbm, scr_hbm, in0, in1, out0, out1,
                  sem_i, sem_o):
        wid = lax.axis_index("s") * NC + lax.axis_index("c")
        nblk = 244 + jnp.where(wid < NB % NW, 1, 0)
        ins = (in0, in1)
        outs = (out0, out1)
        lanes = lax.iota(jnp.int32, 16)

        def fire_in(bi, buf):
            v0 = (wid + NW * bi) * 128
            pltpu.make_async_copy(
                tab_hbm.at[:, pl.ds(v0, 128)], buf, sem_i).start()

        def wait_in(buf):
            pltpu.make_async_copy(
                tab_hbm.at[:, pl.ds(0, 128)], buf, sem_i).wait()

        def fire_out(bi, buf):
            b = wid + NW * bi
            pltpu.make_async_copy(
                buf, scr_hbm.at[pl.ds(64 * b, 64), :], sem_o).start()

        def wait_out(buf):
            pltpu.make_async_copy(
                buf, scr_hbm.at[pl.ds(0, 64), :], sem_o).wait()

        row_idx = [8 * m + (lanes >> 1) for m in range(8)]
        colbase = (lanes & 1) << 6

        def transpose_block(src, dst):
            @plsc.parallel_loop(0, 64, unroll=4)
            def _(e):
                cole = colbase + e
                for m in range(8):
                    vec = src[e, pl.ds(16 * m, 16)]
                    plsc.store_scatter(dst, [row_idx[m], cole], vec)

        fire_in(0, in0)
        fire_in(1, in1)

        def outer(ii, carry):
            for p in range(2):
                bi = 2 * ii + p
                wait_in(ins[p])

                @pl.when(ii >= 1)
                def _():
                    wait_out(outs[p])

                transpose_block(ins[p], outs[p])
                fire_out(bi, outs[p])

                @pl.when(bi + 2 < nblk)
                def _():
                    fire_in(bi + 2, ins[p])

            return carry

        lax.fori_loop(0, 122, outer, 0)

        @pl.when(nblk == 245)
        def _():
            wait_in(ins[0])
            wait_out(outs[0])
            transpose_block(ins[0], outs[0])
            fire_out(244, outs[0])

        wait_out(outs[1])
        wait_out(outs[0])

        @pl.when(wid == 5)
        def _():
            pltpu.sync_copy(tail_hbm, out0.at[pl.ds(0, TAIL // 2), :])
            pltpu.sync_copy(out0.at[pl.ds(0, TAIL // 2), :],
                            scr_hbm.at[pl.ds(64 * NB, TAIL // 2), :])

    return tr_kernel


def _make_gather_ln(B, S, VP):
    ROWS = B * S
    RPW = ROWS // NW
    NSEQ = RPW // S
    NSTR = S // STREAM
    SP = S // 2

    @functools.partial(
        pl.kernel,
        out_type=jax.ShapeDtypeStruct((SP, B, 128), jnp.float32),
        mesh=_mesh(),
        compiler_params=_SC_PARAMS,
        scratch_types=[
            pltpu.VMEM((RPW,), jnp.int32),
            pltpu.VMEM((RPW,), jnp.int32),
            pltpu.VMEM((RPW + 16,), jnp.int32),
            pltpu.VMEM((S, EMBED), jnp.float32),
            pltpu.VMEM((EMBED,), jnp.float32),
            pltpu.VMEM((EMBED,), jnp.float32),
            pltpu.VMEM((S, 128), jnp.float32),
            pltpu.VMEM((S, 128), jnp.float32),
            pltpu.VMEM((SP, 128), jnp.float32),
            pltpu.VMEM((SP, 128), jnp.float32),
            pltpu.SemaphoreType.DMA,
            pltpu.SemaphoreType.DMA,
        ],
    )
    def gl_kernel(x_hbm, scr_hbm, gam_hbm, bet_hbm, poe_hbm, y_hbm,
                  raw_v, idx_v, off_v, poe_v, gam_v, bet_v,
                  g0, g1, o0, o1, sem_g, sem_o):
        wid = lax.axis_index("s") * NC + lax.axis_index("c")
        base_tok = wid * RPW
        base_seq = wid * NSEQ
        pltpu.sync_copy(x_hbm.at[pl.ds(base_tok, RPW)], raw_v)
        pltpu.sync_copy(poe_hbm, poe_v)
        pltpu.sync_copy(gam_hbm, gam_v)
        pltpu.sync_copy(bet_hbm, bet_v)

        def prep(i, carry):
            v = raw_v[pl.ds(16 * i, 16)]
            idx_v[pl.ds(16 * i, 16)] = v >> 1
            off_v[pl.ds(16 * i, 16)] = (v & 1) << 6
            return carry
        lax.fori_loop(0, RPW // 16, prep, 0)

        gam = [gam_v[pl.ds(16 * j, 16)] for j in range(NREG)]
        bet = [bet_v[pl.ds(16 * j, 16)] for j in range(NREG)]
        lanes = lax.iota(jnp.int32, 16)
        perms = [lanes ^ k for k in (1, 2, 4, 8)]
        magic = jnp.full((16,), 0x5F3759DF, jnp.int32)
        gbuf = (g0, g1)
        obuf = (o0, o1)

        def allsum(v):
            for p in perms:
                v = v + v.at[p].get(mode="promise_in_bounds")
            return v

        def fire_gather(c, g):
            for k in range(NSTR):
                pltpu.make_async_copy(
                    scr_hbm.at[idx_v.at[pl.ds(c * S + k * STREAM, STREAM)]],
                    g.at[pl.ds(k * STREAM, STREAM), :], sem_g).start()

        def wait_gather(g):
            for k in range(NSTR):
                pltpu.make_async_copy(
                    scr_hbm.at[idx_v.at[pl.ds(0, STREAM)]],
                    g.at[pl.ds(k * STREAM, STREAM), :], sem_g).wait()

        def fire_out(c, o):
            pltpu.make_async_copy(o, y_hbm.at[:, base_seq + c, :], sem_o).start()

        def wait_out(o):
            pltpu.make_async_copy(o, y_hbm.at[:, 0, :], sem_o).wait()

        def compute(c, g, o):
            @plsc.parallel_loop(0, S // 8, unroll=2)
            def group(gi):
                off16 = off_v[pl.ds(c * S + 8 * gi, 16)]
                for u in range(8):
                    r = 8 * gi + u
                    off = off16[u]
                    x = [g[r, pl.ds(off + 16 * j, 16)]
                         + poe_v[r, pl.ds(16 * j, 16)] for j in range(NREG)]
                    tot = allsum((x[0] + x[1]) + (x[2] + x[3]))
                    tot2 = allsum((x[0] * x[0] + x[1] * x[1])
                                  + (x[2] * x[2] + x[3] * x[3]))
                    mean = tot * (1.0 / EMBED)
                    v = tot2 * (1.0 / EMBED) - mean * mean + 1e-5
                    iv = plsc.bitcast(v, jnp.int32)
                    y = plsc.bitcast(magic - (iv >> 1), jnp.float32)
                    for _ in range(3):
                        y = y * (1.5 - 0.5 * v * y * y)
                    orow = 4 * gi + u // 2
                    ocol = (u % 2) * 64
                    for j in range(NREG):
                        a = gam[j] * y
                        b = bet[j] - mean * a
                        o[orow, pl.ds(ocol + 16 * j, 16)] = x[j] * a + b

        fire_gather(0, g0)
        fire_gather(1, g1)

        def outer(cc, carry):
            for p in range(2):
                c = 2 * cc + p
                wait_gather(gbuf[p])

                @pl.when(cc >= 1)
                def _():
                    wait_out(obuf[p])

                compute(c, gbuf[p], obuf[p])
                fire_out(c, obuf[p])

                @pl.when(cc < NSEQ // 2 - 1)
                def _():
                    fire_gather(c + 2, gbuf[p])

            return carry

        lax.fori_loop(0, NSEQ // 2, outer, 0)
        wait_out(o0)
        wait_out(o1)

    return gl_kernel


def _make_finish(B, S):
    SP = S // 2

    def fin_kernel(y_ref, o_ref):
        for q in range(4):
            y = y_ref[q]
            a = y[:, 0:EMBED].T
            b = y[:, EMBED:128].T
            for t in range(T):
                o_ref[t, 2 * q] = a
                o_ref[t, 2 * q + 1] = b

    return pl.pallas_call(
        fin_kernel,
        grid=(SP // 4,),
        in_specs=[pl.BlockSpec((4, B, 128), lambda i: (i, 0, 0))],
        out_specs=pl.BlockSpec((T, 8, EMBED, B), lambda i: (0, i, 0, 0)),
        out_shape=jax.ShapeDtypeStruct((T, S, EMBED, B), jnp.float32),
    )


def kernel(x, emb_table, ln_gamma, ln_beta, poe):
    B, S = x.shape
    V = emb_table.shape[0]
    tab_t = emb_table.T
    nb = V // 128
    tail = emb_table[nb * 128:].reshape(-1, 128)
    xf = x.astype(jnp.int32).reshape(-1)
    scratch = _make_transpose(V)(tab_t, tail)
    y = _make_gather_ln(B, S, V // 2)(
        xf, scratch, ln_gamma, ln_beta, poe[:S])
    p = _make_finish(B, S)(y)
    return jnp.transpose(p, (0, 3, 1, 2))
